# baseline (speedup 1.0000x reference)
import functools

import jax
import jax.numpy as jnp
from jax import lax
from jax.experimental import pallas as pl
from jax.experimental.pallas import tpu as pltpu
from jax.experimental.pallas import tpu_sc as plsc

N_NODES_ = 10000
N_EDGES_ = 320000
D_ = 128
DP_ = D_ // 2
NUM_REL_ = 64

NC_ = 2
NS_ = 16
NW_ = NC_ * NS_
E_PER_W_ = N_EDGES_ // NW_
B_ = 80
BP_ = B_ + 16
NBATCH_ = E_PER_W_ // B_


def _sc_body(z_hbm, src_hbm, dst_hbm, typ_hbm, rel_hbm, out_hbm,
             src_v, dst_v, typ_v, s0_v, o0_v, s1_v, o1_v, rel_v, out_v,
             sem0, sem1):
    wid = lax.axis_index("c") * NS_ + lax.axis_index("s")

    pltpu.sync_copy(src_hbm.at[wid], src_v)
    pltpu.sync_copy(dst_hbm.at[wid], dst_v)
    pltpu.sync_copy(typ_hbm.at[wid], typ_v)
    pltpu.sync_copy(rel_hbm, rel_v)

    bufs = ((s0_v, o0_v, sem0), (s1_v, o1_v, sem1))

    def issue(i, b):
        s_v, o_v, sem = bufs[b]
        pltpu.async_copy(z_hbm.at[src_v.at[i]], s_v, sem)
        pltpu.async_copy(z_hbm.at[dst_v.at[i]], o_v, sem)

    def wait(b):
        s_v, o_v, sem = bufs[b]
        pltpu.make_async_copy(z_hbm.at[src_v.at[0]], s_v, sem).wait()
        pltpu.make_async_copy(z_hbm.at[dst_v.at[0]], o_v, sem).wait()

    lane = lax.iota(jnp.int32, 16)

    def compute(i, b):
        s_v, o_v, _ = bufs[b]

        def group_body(g, _):
            eb = g * 16
            tvec = typ_v[i, pl.ds(eb, 16)]
            for sub in range(16):
                e = eb + sub
                t = tvec[sub]
                acc0 = jnp.zeros((16,), jnp.float32)
                acc1 = jnp.zeros((16,), jnp.float32)
                for j in range(DP_ // 16):
                    sl = pl.ds(16 * j, 16)
                    prod = (plsc.bitcast(s_v[e, sl], jnp.bfloat16)
                            * plsc.bitcast(o_v[e, sl], jnp.bfloat16)
                            * plsc.bitcast(rel_v[t, sl], jnp.bfloat16))
                    pa, pb = plsc.unpack(prod,
                                         format=plsc.PackFormat.INTERLEAVED,
                                         preferred_element_type=jnp.float32)
                    acc0 = acc0 + pa
                    acc1 = acc1 + pb
                total = plsc.cumsum(acc0 + acc1)
                plsc.store_compressed(out_v.at[i, pl.ds(e, 16)], total,
                                      mask=lane == 15)
            return 0

        lax.fori_loop(0, B_ // 16, group_body, 0)

    issue(0, 0)

    def pair_body(p, _):
        i0 = p * 2
        issue(i0 + 1, 1)
        wait(0)
        compute(i0, 0)
        issue(i0 + 2, 0)
        wait(1)
        compute(i0 + 1, 1)
        return 0

    lax.fori_loop(0, (NBATCH_ - 1) // 2, pair_body, 0)
    wait(0)
    compute(NBATCH_ - 1, 0)

    pltpu.sync_copy(out_v, out_hbm.at[wid])


@jax.jit
def _dist_mult_sc(zp, src, dst, typ, relp):
    mesh = plsc.VectorSubcoreMesh(core_axis_name="c", subcore_axis_name="s")
    f = pl.kernel(
        _sc_body,
        out_type=jax.ShapeDtypeStruct((NW_, NBATCH_, BP_), jnp.float32),
        mesh=mesh,
        scratch_types=[
            pltpu.VMEM((NBATCH_, B_), jnp.int32),
            pltpu.VMEM((NBATCH_, B_), jnp.int32),
            pltpu.VMEM((NBATCH_, B_), jnp.int32),
            pltpu.VMEM((B_, DP_), jnp.uint32),
            pltpu.VMEM((B_, DP_), jnp.uint32),
            pltpu.VMEM((B_, DP_), jnp.uint32),
            pltpu.VMEM((B_, DP_), jnp.uint32),
            pltpu.VMEM((NUM_REL_, DP_), jnp.uint32),
            pltpu.VMEM((NBATCH_, BP_), jnp.float32),
            pltpu.SemaphoreType.DMA,
            pltpu.SemaphoreType.DMA,
        ],
        compiler_params=pltpu.CompilerParams(needs_layout_passes=False,
                                             use_tc_tiling_on_sc=False),
    )
    return f(zp, src, dst, typ, relp)[:, :, :B_].reshape(N_EDGES_)


def _pack_rows(x):
    xb = x.astype(jnp.bfloat16)
    return lax.bitcast_convert_type(
        xb.reshape(x.shape[0], x.shape[1] // 2, 2), jnp.uint32)


def kernel(z, edge_index, edge_type, relation_embedding):
    src = edge_index[0].astype(jnp.int32).reshape(NW_, NBATCH_, B_)
    dst = edge_index[1].astype(jnp.int32).reshape(NW_, NBATCH_, B_)
    typ = edge_type.astype(jnp.int32).reshape(NW_, NBATCH_, B_)
    return _dist_mult_sc(_pack_rows(z), src, dst, typ,
                         _pack_rows(relation_embedding))


# speedup vs baseline: 1.6054x; 1.6054x over previous
import functools

import jax
import jax.numpy as jnp
from jax import lax
from jax.experimental import pallas as pl
from jax.experimental.pallas import tpu as pltpu
from jax.experimental.pallas import tpu_sc as plsc

N_NODES_ = 10000
N_EDGES_ = 320000
D_ = 128
DP_ = D_ // 2
NUM_REL_ = 64

NC_ = 2
NS_ = 16
NW_ = NC_ * NS_
E_PER_W_ = N_EDGES_ // NW_
B_ = 80
NBATCH_ = E_PER_W_ // B_


def _sc_body(z_hbm, src_hbm, dst_hbm, typ_hbm, rel_hbm, out_hbm,
             src_v, dst_v, typ_v, s0_v, o0_v, s1_v, o1_v, rel_v, out_v,
             sem0, sem1):
    wid = lax.axis_index("c") * NS_ + lax.axis_index("s")

    pltpu.sync_copy(src_hbm.at[wid], src_v)
    pltpu.sync_copy(dst_hbm.at[wid], dst_v)
    pltpu.sync_copy(typ_hbm.at[wid], typ_v)
    pltpu.sync_copy(rel_hbm, rel_v)

    bufs = ((s0_v, o0_v, sem0), (s1_v, o1_v, sem1))

    def issue(i, b):
        s_v, o_v, sem = bufs[b]
        pltpu.async_copy(z_hbm.at[src_v.at[i]], s_v, sem)
        pltpu.async_copy(z_hbm.at[dst_v.at[i]], o_v, sem)

    def wait(b):
        s_v, o_v, sem = bufs[b]
        pltpu.make_async_copy(z_hbm.at[src_v.at[0]], s_v, sem).wait()
        pltpu.make_async_copy(z_hbm.at[dst_v.at[0]], o_v, sem).wait()

    lane = lax.iota(jnp.int32, 16)

    def compute(i, b):
        s_v, o_v, _ = bufs[b]

        def group_body(g, _):
            eb = g * 16
            tvec = typ_v[i, pl.ds(eb, 16)]
            score = jnp.zeros((16,), jnp.float32)
            for sub in range(16):
                e = eb + sub
                t = tvec[sub]
                acc0 = jnp.zeros((16,), jnp.float32)
                acc1 = jnp.zeros((16,), jnp.float32)
                for j in range(DP_ // 16):
                    sl = pl.ds(16 * j, 16)
                    prod = (plsc.bitcast(s_v[e, sl], jnp.bfloat16)
                            * plsc.bitcast(o_v[e, sl], jnp.bfloat16)
                            * plsc.bitcast(rel_v[t, sl], jnp.bfloat16))
                    pa, pb = plsc.unpack(prod,
                                         format=plsc.PackFormat.INTERLEAVED,
                                         preferred_element_type=jnp.float32)
                    acc0 = acc0 + pa
                    acc1 = acc1 + pb
                score = jnp.where(lane == sub, jnp.sum(acc0 + acc1), score)
            out_v[i, pl.ds(eb, 16)] = score
            return 0

        lax.fori_loop(0, B_ // 16, group_body, 0)

    issue(0, 0)

    def pair_body(p, _):
        i0 = p * 2
        issue(i0 + 1, 1)
        wait(0)
        compute(i0, 0)
        issue(i0 + 2, 0)
        wait(1)
        compute(i0 + 1, 1)
        return 0

    lax.fori_loop(0, (NBATCH_ - 1) // 2, pair_body, 0)
    wait(0)
    compute(NBATCH_ - 1, 0)

    pltpu.sync_copy(out_v, out_hbm.at[wid])


@jax.jit
def _dist_mult_sc(zp, src, dst, typ, relp):
    mesh = plsc.VectorSubcoreMesh(core_axis_name="c", subcore_axis_name="s")
    f = pl.kernel(
        _sc_body,
        out_type=jax.ShapeDtypeStruct((NW_, NBATCH_, B_), jnp.float32),
        mesh=mesh,
        scratch_types=[
            pltpu.VMEM((NBATCH_, B_), jnp.int32),
            pltpu.VMEM((NBATCH_, B_), jnp.int32),
            pltpu.VMEM((NBATCH_, B_), jnp.int32),
            pltpu.VMEM((B_, DP_), jnp.uint32),
            pltpu.VMEM((B_, DP_), jnp.uint32),
            pltpu.VMEM((B_, DP_), jnp.uint32),
            pltpu.VMEM((B_, DP_), jnp.uint32),
            pltpu.VMEM((NUM_REL_, DP_), jnp.uint32),
            pltpu.VMEM((NBATCH_, B_), jnp.float32),
            pltpu.SemaphoreType.DMA,
            pltpu.SemaphoreType.DMA,
        ],
        compiler_params=pltpu.CompilerParams(needs_layout_passes=False,
                                             use_tc_tiling_on_sc=False),
    )
    return f(zp, src, dst, typ, relp).reshape(N_EDGES_)


def _pack_rows(x):
    xb = x.astype(jnp.bfloat16)
    return lax.bitcast_convert_type(
        xb.reshape(x.shape[0], x.shape[1] // 2, 2), jnp.uint32)


def kernel(z, edge_index, edge_type, relation_embedding):
    src = edge_index[0].astype(jnp.int32).reshape(NW_, NBATCH_, B_)
    dst = edge_index[1].astype(jnp.int32).reshape(NW_, NBATCH_, B_)
    typ = edge_type.astype(jnp.int32).reshape(NW_, NBATCH_, B_)
    return _dist_mult_sc(_pack_rows(z), src, dst, typ,
                         _pack_rows(relation_embedding))
